# baseline (speedup 1.0000x reference)
import functools

import jax
import jax.numpy as jnp
from jax.experimental import pallas as pl
from jax.experimental.pallas import tpu as pltpu

_CHUNK = 4096
_RING = 6
_PREFETCH = 3


def _ring_body(B, cap, dim, feat_ref, mem_ref, fit_ref,
               out_mem_ref, out_fit_ref, *scratch):
    bufs = scratch[:_RING]
    fit_in = scratch[_RING]
    ones_v = scratch[_RING + 1]
    sin = scratch[_RING + 2:_RING + 2 + _RING]
    sout = scratch[_RING + 2 + _RING:2 * _RING + 2 + _RING]
    sem_fi, sem_fo = scratch[-2:]

    n_chunks = (cap + _CHUNK - 1) // _CHUNK
    nf = B // _CHUNK
    tail_n = cap - B

    ones_v[...] = jnp.ones_like(ones_v)

    fit_tail_in = pltpu.make_async_copy(
        fit_ref.at[pl.ds(B, tail_n)], fit_in, sem_fi)
    fit_tail_in.start()
    ones_out = pltpu.make_async_copy(
        ones_v, out_fit_ref.at[pl.ds(0, B)], sem_fo)
    ones_out.start()
    fit_tail_in.wait()
    fit_tail_out = pltpu.make_async_copy(
        fit_in, out_fit_ref.at[pl.ds(B, tail_n)], sem_fo)
    fit_tail_out.start()

    def rows(c):
        return min(_CHUNK, cap - c * _CHUNK)

    def in_copy(c, s):
        src = feat_ref if c < nf else mem_ref
        return pltpu.make_async_copy(
            src.at[pl.ds(c * _CHUNK, rows(c))],
            bufs[s].at[pl.ds(0, rows(c))], sin[s])

    def out_copy(c, s):
        return pltpu.make_async_copy(
            bufs[s].at[pl.ds(0, rows(c))],
            out_mem_ref.at[pl.ds(c * _CHUNK, rows(c))], sout[s])

    for c in range(min(_PREFETCH, n_chunks)):
        in_copy(c, c % _RING).start()
    for k in range(n_chunks):
        s = k % _RING
        in_copy(k, s).wait()
        out_copy(k, s).start()
        c = k + _PREFETCH
        if c < n_chunks:
            sp = c % _RING
            if c - _RING >= 0:
                out_copy(c - _RING, sp).wait()
            in_copy(c, sp).start()
    for c in range(max(0, n_chunks - _RING), n_chunks):
        out_copy(c, c % _RING).wait()
    ones_out.wait()
    fit_tail_out.wait()


def kernel(features, memory, fitness):
    B = features.shape[0]
    cap, dim = memory.shape
    scratch = (
        [pltpu.VMEM((_CHUNK, dim), memory.dtype) for _ in range(_RING)]
        + [pltpu.VMEM((cap - B,), fitness.dtype)]
        + [pltpu.VMEM((B,), fitness.dtype)]
        + [pltpu.SemaphoreType.DMA for _ in range(2 * _RING + 2)]
    )
    out_mem, out_fit = pl.pallas_call(
        functools.partial(_ring_body, B, cap, dim),
        out_shape=(
            jax.ShapeDtypeStruct((cap, dim), memory.dtype),
            jax.ShapeDtypeStruct((cap,), fitness.dtype),
        ),
        in_specs=[
            pl.BlockSpec(memory_space=pl.ANY),
            pl.BlockSpec(memory_space=pl.ANY),
            pl.BlockSpec(memory_space=pl.ANY),
        ],
        out_specs=(
            pl.BlockSpec(memory_space=pl.ANY),
            pl.BlockSpec(memory_space=pl.ANY),
        ),
        scratch_shapes=scratch,
    )(features, memory, fitness)
    return out_mem, out_fit


# speedup vs baseline: 1.0139x; 1.0139x over previous
import functools

import jax
import jax.numpy as jnp
from jax.experimental import pallas as pl

_BLOCK = 8192


def _emb_write_body(nf, feat_ref, mem_ref, fit_ref, out_mem_ref, out_fit_ref):
    i = pl.program_id(0)

    @pl.when(i < nf)
    def _():
        out_mem_ref[...] = feat_ref[...]
        out_fit_ref[...] = jnp.ones_like(out_fit_ref)

    @pl.when(i >= nf)
    def _():
        out_mem_ref[...] = mem_ref[...]
        out_fit_ref[...] = fit_ref[...]


def kernel(features, memory, fitness):
    B = features.shape[0]
    cap, dim = memory.shape
    block = _BLOCK if B % _BLOCK == 0 else 2048
    nf = B // block
    grid = (cap + block - 1) // block

    def feat_map(i):
        return (jnp.minimum(i, nf - 1), 0)

    def mem_map(i):
        return (jnp.maximum(i, nf), 0)

    def fit_map(i):
        return (jnp.maximum(i, nf),)

    out_mem, out_fit = pl.pallas_call(
        functools.partial(_emb_write_body, nf),
        grid=(grid,),
        out_shape=(
            jax.ShapeDtypeStruct((cap, dim), memory.dtype),
            jax.ShapeDtypeStruct((cap,), fitness.dtype),
        ),
        in_specs=[
            pl.BlockSpec((block, dim), feat_map),
            pl.BlockSpec((block, dim), mem_map),
            pl.BlockSpec((block,), fit_map),
        ],
        out_specs=(
            pl.BlockSpec((block, dim), lambda i: (i, 0)),
            pl.BlockSpec((block,), lambda i: (i,)),
        ),
    )(features, memory, fitness)
    return out_mem, out_fit
